# baseline (speedup 1.0000x reference)
import functools

import jax
import jax.numpy as jnp
from jax import lax
from jax.experimental import pallas as pl
from jax.experimental.pallas import tpu as pltpu
from jax.experimental.pallas import tpu_sc as plsc

_B = 4096
_D = 32

_info = plsc.get_sparse_core_info()
_NC, _NS = _info.num_cores, _info.num_subcores
_NW = _NC * _NS
_BPW = _B // _NW


def _make_gather():
    mesh = plsc.VectorSubcoreMesh(core_axis_name="c", subcore_axis_name="s")

    @functools.partial(
        pl.kernel,
        mesh=mesh,
        compiler_params=pltpu.CompilerParams(use_tc_tiling_on_sc=False),
        out_type=(
            jax.ShapeDtypeStruct((_B, _D), jnp.float32),
            jax.ShapeDtypeStruct((_B, _D), jnp.float32),
        ),
        scratch_types=[
            pltpu.VMEM((_BPW,), jnp.int32),
            pltpu.VMEM((_BPW,), jnp.int32),
            pltpu.VMEM((_BPW, _D), jnp.float32),
            pltpu.VMEM((_BPW, _D), jnp.float32),
            pltpu.SemaphoreType.DMA,
            pltpu.SemaphoreType.DMA,
        ],
    )
    def gather_k(tgt_hbm, ctx_hbm, in_hbm, out_hbm, a_out, b_out,
                 idx_a, idx_b, rows_a, rows_b, sem_a, sem_b):
        wid = lax.axis_index("s") * _NC + lax.axis_index("c")
        base = wid * _BPW
        pltpu.sync_copy(tgt_hbm.at[pl.ds(base, _BPW)], idx_a)
        pltpu.sync_copy(ctx_hbm.at[pl.ds(base, _BPW)], idx_b)
        ca = pltpu.async_copy(in_hbm.at[idx_a], rows_a, sem_a)
        cb = pltpu.async_copy(out_hbm.at[idx_b], rows_b, sem_b)
        ca.wait()
        cb.wait()
        pltpu.sync_copy(rows_a, a_out.at[pl.ds(base, _BPW)])
        pltpu.sync_copy(rows_b, b_out.at[pl.ds(base, _BPW)])

    return gather_k


_gather = _make_gather()

_BM = 512


def _mm_body(a_ref, b_ref, o_ref):
    o_ref[...] = lax.dot_general(
        a_ref[...], b_ref[...],
        (((1,), (1,)), ((), ())),
        preferred_element_type=jnp.float32,
    )


_matmul = pl.pallas_call(
    _mm_body,
    grid=(_B // _BM,),
    in_specs=[
        pl.BlockSpec((_BM, _D), lambda i: (i, 0)),
        pl.BlockSpec((_B, _D), lambda i: (0, 0)),
    ],
    out_specs=pl.BlockSpec((_BM, _B), lambda i: (i, 0)),
    out_shape=jax.ShapeDtypeStruct((_B, _B), jnp.float32),
)


def kernel(target, context, in_embed, out_embed):
    in_rows, ctx_rows = _gather(
        target.astype(jnp.int32), context.astype(jnp.int32), in_embed, out_embed
    )
    return _matmul(in_rows, ctx_rows)


# speedup vs baseline: 8.4140x; 8.4140x over previous
import functools

import jax
import jax.numpy as jnp
from jax import lax
from jax.experimental import pallas as pl
from jax.experimental.pallas import tpu as pltpu
from jax.experimental.pallas import tpu_sc as plsc

_B = 4096
_D = 32
_V = 1000000

_info = plsc.get_sparse_core_info()
_NC, _NS = _info.num_cores, _info.num_subcores
_NW = _NC * _NS
_BPW = _B // _NW
_G = 16


def _make_gather():
    mesh = plsc.VectorSubcoreMesh(core_axis_name="c", subcore_axis_name="s")

    @functools.partial(
        pl.kernel,
        mesh=mesh,
        compiler_params=pltpu.CompilerParams(use_tc_tiling_on_sc=True, needs_layout_passes=False),
        out_type=(
            jax.ShapeDtypeStruct((_D, _B), jnp.float32),
            jax.ShapeDtypeStruct((_D, _B), jnp.float32),
        ),
        scratch_types=[
            pltpu.VMEM((_BPW,), jnp.int32),
            pltpu.VMEM((_BPW,), jnp.int32),
            pltpu.VMEM((_D, _BPW), jnp.float32),
            pltpu.VMEM((_D, _BPW), jnp.float32),
            pltpu.VMEM((_G, _D, 128), jnp.float32),
            pltpu.SemaphoreType.DMA,
        ],
    )
    def gather_k(tgt_hbm, ctx_hbm, inT_hbm, outT_hbm, aT_out, bT_out,
                 idx_a, idx_b, at_v, bt_v, slab, sem):
        wid = lax.axis_index("s") * _NC + lax.axis_index("c")
        base = pl.multiple_of(wid * _BPW, 128)
        pltpu.sync_copy(tgt_hbm.at[pl.ds(base, _BPW)], idx_a)
        pltpu.sync_copy(ctx_hbm.at[pl.ds(base, _BPW)], idx_b)
        row_lo = lax.iota(jnp.int32, 16)
        row_hi = row_lo + 16

        def phase(idx_ref, src_ref, dst_ref):
            def group(g, carry):
                j0 = g * _G
                vv = idx_ref[pl.ds(j0, _G)]
                copies = []
                for b in range(_G):
                    l128 = pl.multiple_of((vv[b] >> 7) * 128, 128)
                    copies.append(pltpu.async_copy(
                        src_ref.at[:, pl.ds(l128, 128)], slab.at[b], sem))
                for c in copies:
                    c.wait()
                for b in range(_G):
                    col = jnp.full((16,), vv[b] & 127, jnp.int32)
                    jv = jnp.full((16,), j0 + b, jnp.int32)
                    lo = plsc.load_gather(slab.at[b], [row_lo, col])
                    hi = plsc.load_gather(slab.at[b], [row_hi, col])
                    plsc.store_scatter(dst_ref, [row_lo, jv], lo)
                    plsc.store_scatter(dst_ref, [row_hi, jv], hi)
                return carry

            lax.fori_loop(0, _BPW // _G, group, 0)

        phase(idx_a, inT_hbm, at_v)
        phase(idx_b, outT_hbm, bt_v)
        pltpu.sync_copy(at_v, aT_out.at[:, pl.ds(base, _BPW)])
        pltpu.sync_copy(bt_v, bT_out.at[:, pl.ds(base, _BPW)])

    return gather_k


_gather = _make_gather()

_BM = 512


def _mm_body(a_ref, b_ref, o_ref):
    o_ref[...] = lax.dot_general(
        a_ref[...], b_ref[...],
        (((0,), (0,)), ((), ())),
        preferred_element_type=jnp.float32,
    )


_matmul = pl.pallas_call(
    _mm_body,
    grid=(_B // _BM,),
    in_specs=[
        pl.BlockSpec((_D, _BM), lambda i: (0, i)),
        pl.BlockSpec((_D, _B), lambda i: (0, 0)),
    ],
    out_specs=pl.BlockSpec((_BM, _B), lambda i: (i, 0)),
    out_shape=jax.ShapeDtypeStruct((_B, _B), jnp.float32),
)


def kernel(target, context, in_embed, out_embed):
    aT, bT = _gather(
        target.astype(jnp.int32), context.astype(jnp.int32),
        in_embed.T, out_embed.T,
    )
    return _matmul(aT, bT)


# speedup vs baseline: 8.5861x; 1.0205x over previous
import functools

import jax
import jax.numpy as jnp
from jax import lax
from jax.experimental import pallas as pl
from jax.experimental.pallas import tpu as pltpu
from jax.experimental.pallas import tpu_sc as plsc

_B = 4096
_D = 32
_V = 1000000

_info = plsc.get_sparse_core_info()
_NC, _NS = _info.num_cores, _info.num_subcores
_NW = _NC * _NS
_BPW = _B // _NW
_G = 16


def _make_gather():
    mesh = plsc.VectorSubcoreMesh(core_axis_name="c", subcore_axis_name="s")

    @functools.partial(
        pl.kernel,
        mesh=mesh,
        compiler_params=pltpu.CompilerParams(use_tc_tiling_on_sc=True, needs_layout_passes=False),
        out_type=(
            jax.ShapeDtypeStruct((_D, _B), jnp.float32),
            jax.ShapeDtypeStruct((_D, _B), jnp.float32),
        ),
        scratch_types=[
            pltpu.VMEM((_BPW,), jnp.int32),
            pltpu.VMEM((_BPW,), jnp.int32),
            pltpu.VMEM((_D, _BPW), jnp.float32),
            pltpu.VMEM((_D, _BPW), jnp.float32),
            pltpu.VMEM((_G, _D, 128), jnp.float32),
            pltpu.SemaphoreType.DMA,
            pltpu.SemaphoreType.DMA,
        ],
    )
    def gather_k(tgt_hbm, ctx_hbm, inT_hbm, outT_hbm, aT_out, bT_out,
                 idx_a, idx_b, at_v, bt_v, slab, sem_a, sem_b):
        wid = lax.axis_index("s") * _NC + lax.axis_index("c")
        base = pl.multiple_of(wid * _BPW, 128)
        pltpu.sync_copy(tgt_hbm.at[pl.ds(base, _BPW)], idx_a)
        pltpu.sync_copy(ctx_hbm.at[pl.ds(base, _BPW)], idx_b)
        row_lo = lax.iota(jnp.int32, 16)
        row_hi = row_lo + 16
        n_groups = _BPW // _G

        def phase(idx_ref, src_ref, dst_ref):
            def issue(vb, slot, sem):
                l128 = pl.multiple_of((vb >> 7) * 128, 128)
                pltpu.async_copy(src_ref.at[:, pl.ds(l128, 128)], slab.at[slot], sem)

            def extract(vb, j, slot):
                col = jnp.full((16,), vb & 127, jnp.int32)
                jv = jnp.full((16,), j, jnp.int32)
                lo = plsc.load_gather(slab.at[slot], [row_lo, col])
                hi = plsc.load_gather(slab.at[slot], [row_hi, col])
                plsc.store_scatter(dst_ref, [row_lo, jv], lo)
                plsc.store_scatter(dst_ref, [row_hi, jv], hi)

            vv0 = idx_ref[pl.ds(0, _G)]
            for b in range(8):
                issue(vv0[b], b, sem_a)
            for b in range(8, 16):
                issue(vv0[b], b, sem_b)

            def group(g, vcur):
                nxt = jnp.minimum((g + 1) * _G, _BPW - _G)
                vnxt = idx_ref[pl.ds(nxt, _G)]
                not_last = g < n_groups - 1
                for half, sem in ((0, sem_a), (1, sem_b)):
                    for b in range(half * 8, half * 8 + 8):
                        pltpu.make_async_copy(
                            src_ref.at[:, pl.ds(0, 128)], slab.at[b], sem).wait()
                    for b in range(half * 8, half * 8 + 8):
                        extract(vcur[b], g * _G + b, b)

                    @pl.when(not_last)
                    def _():
                        for b in range(half * 8, half * 8 + 8):
                            issue(vnxt[b], b, sem)
                return vnxt

            lax.fori_loop(0, n_groups, group, vv0)

        phase(idx_a, inT_hbm, at_v)
        phase(idx_b, outT_hbm, bt_v)
        pltpu.sync_copy(at_v, aT_out.at[:, pl.ds(base, _BPW)])
        pltpu.sync_copy(bt_v, bT_out.at[:, pl.ds(base, _BPW)])

    return gather_k


_gather = _make_gather()

_BM = 1024


def _mm_body(a_ref, b_ref, o_ref):
    o_ref[...] = lax.dot_general(
        a_ref[...], b_ref[...],
        (((0,), (0,)), ((), ())),
        preferred_element_type=jnp.float32,
    )


_matmul = pl.pallas_call(
    _mm_body,
    grid=(_B // _BM,),
    in_specs=[
        pl.BlockSpec((_D, _BM), lambda i: (0, i)),
        pl.BlockSpec((_D, _B), lambda i: (0, 0)),
    ],
    out_specs=pl.BlockSpec((_BM, _B), lambda i: (i, 0)),
    out_shape=jax.ShapeDtypeStruct((_B, _B), jnp.float32),
)


def kernel(target, context, in_embed, out_embed):
    aT, bT = _gather(
        target.astype(jnp.int32), context.astype(jnp.int32),
        in_embed.T, out_embed.T,
    )
    return _matmul(aT, bT)
